# baseline (speedup 1.0000x reference)
import functools

import jax
import jax.numpy as jnp
from jax.experimental import pallas as pl
from jax.experimental.pallas import tpu as pltpu


def _bias_kernel(w_ref, le_ref, b_ref, bias_ref):
    bias_ref[...] = jnp.sum(w_ref[...] * le_ref[...], axis=1) + b_ref[...]


def _matmul_kernel(x_ref, w_ref, bias_ref, o_ref):
    x_bf = x_ref[...].astype(jnp.bfloat16)
    acc = jax.lax.dot_general(
        x_bf, w_ref[...],
        dimension_numbers=(((1,), (1,)), ((), ())),
        preferred_element_type=jnp.float32,
    )
    o_ref[...] = acc + bias_ref[...]


@functools.partial(jax.jit, static_argnames=())
def kernel(bert_output, label_embed, W, b):
    B, H = bert_output.shape
    L = W.shape[0]

    lc = L // 2
    label_bias = pl.pallas_call(
        _bias_kernel,
        grid=(2,),
        in_specs=[
            pl.BlockSpec((lc, H), lambda c: (c, 0)),
            pl.BlockSpec((lc, H), lambda c: (c, 0)),
            pl.BlockSpec((lc,), lambda c: (c,)),
        ],
        out_specs=pl.BlockSpec((lc,), lambda c: (c,)),
        out_shape=jax.ShapeDtypeStruct((L,), jnp.float32),
        compiler_params=pltpu.CompilerParams(
            dimension_semantics=("parallel",),
        ),
    )(W, label_embed, b)
    label_bias = label_bias.reshape(1, L)

    W_bf = W.astype(jnp.bfloat16)
    bm = 512
    nb = B // bm // 2

    out = pl.pallas_call(
        _matmul_kernel,
        grid=(2, nb),
        in_specs=[
            pl.BlockSpec((bm, H), lambda c, j, nb=nb: (c * nb + j, 0)),
            pl.BlockSpec((L, H), lambda c, j: (0, 0)),
            pl.BlockSpec((1, L), lambda c, j: (0, 0)),
        ],
        out_specs=pl.BlockSpec((bm, L), lambda c, j, nb=nb: (c * nb + j, 0)),
        out_shape=jax.ShapeDtypeStruct((B, L), jnp.float32),
        compiler_params=pltpu.CompilerParams(
            dimension_semantics=("parallel", "arbitrary"),
        ),
    )(bert_output, W_bf, label_bias)
    return out


# speedup vs baseline: 1.0695x; 1.0695x over previous
import functools

import jax
import jax.numpy as jnp
from jax.experimental import pallas as pl
from jax.experimental.pallas import tpu as pltpu


def _prep_kernel(w_ref, le_ref, b_ref, wbf_ref, bias_ref):
    w = w_ref[...]
    wbf_ref[...] = w.astype(jnp.bfloat16)
    bias_ref[...] = (jnp.sum(w * le_ref[...], axis=1) + b_ref[...])[None, :]


def _matmul_kernel(x_ref, w_ref, bias_ref, o_ref):
    x_bf = x_ref[...].astype(jnp.bfloat16)
    acc = jax.lax.dot_general(
        x_bf, w_ref[...],
        dimension_numbers=(((1,), (1,)), ((), ())),
        preferred_element_type=jnp.float32,
    )
    o_ref[...] = acc + bias_ref[...]


@functools.partial(jax.jit, static_argnames=())
def kernel(bert_output, label_embed, W, b):
    B, H = bert_output.shape
    L = W.shape[0]

    lc = L // 2
    W_bf, label_bias = pl.pallas_call(
        _prep_kernel,
        grid=(2,),
        in_specs=[
            pl.BlockSpec((lc, H), lambda c: (c, 0)),
            pl.BlockSpec((lc, H), lambda c: (c, 0)),
            pl.BlockSpec((lc,), lambda c: (c,)),
        ],
        out_specs=[
            pl.BlockSpec((lc, H), lambda c: (c, 0)),
            pl.BlockSpec((1, lc), lambda c: (0, c)),
        ],
        out_shape=[
            jax.ShapeDtypeStruct((L, H), jnp.bfloat16),
            jax.ShapeDtypeStruct((1, L), jnp.float32),
        ],
        compiler_params=pltpu.CompilerParams(
            dimension_semantics=("parallel",),
        ),
    )(W, label_embed, b)

    bm = 1024
    nb = B // bm // 2

    out = pl.pallas_call(
        _matmul_kernel,
        grid=(2, nb),
        in_specs=[
            pl.BlockSpec((bm, H), lambda c, j, nb=nb: (c * nb + j, 0)),
            pl.BlockSpec((L, H), lambda c, j: (0, 0)),
            pl.BlockSpec((1, L), lambda c, j: (0, 0)),
        ],
        out_specs=pl.BlockSpec((bm, L), lambda c, j, nb=nb: (c * nb + j, 0)),
        out_shape=jax.ShapeDtypeStruct((B, L), jnp.float32),
        compiler_params=pltpu.CompilerParams(
            dimension_semantics=("parallel", "arbitrary"),
            vmem_limit_bytes=100 * 1024 * 1024,
        ),
    )(bert_output, W_bf, label_bias)
    return out
